# baseline (speedup 1.0000x reference)
import functools

import jax
import jax.numpy as jnp
import numpy as np
from jax import lax
from jax.experimental import pallas as pl
from jax.experimental.pallas import tpu as pltpu
from jax.experimental.pallas import tpu_sc as plsc

N = 1048576
NT = 16
NC = 2
C = N // NT
W = 8192
NW = C // W
V = W // 16
MEAN = (N - 1) / 2.0

mesh = plsc.VectorSubcoreMesh(core_axis_name="c", subcore_axis_name="s",
                              num_cores=NC, num_subcores=NT)
_params = pltpu.CompilerParams(needs_layout_passes=False)

_i32 = functools.partial(jax.ShapeDtypeStruct, dtype=jnp.int32)
_f32 = functools.partial(jax.ShapeDtypeStruct, dtype=jnp.float32)


def _u32key(xv):
    b = plsc.bitcast(xv, jnp.int32)
    return jnp.where(b < 0, jnp.bitwise_xor(b, jnp.int32(-1)),
                     jnp.bitwise_or(b, jnp.int32(-(2 ** 31))))


def _digit(kv, shift):
    s = jnp.full((16,), shift, jnp.int32)
    return jnp.bitwise_and(lax.shift_right_logical(kv, s),
                           jnp.full((16,), 255, jnp.int32))


def _make_pass(shift, is_p0, write_keys):
    n_out = 2 if write_keys else 1
    out_type = [_i32((2 * N,))] * n_out

    @functools.partial(
        pl.kernel, mesh=mesh, out_type=out_type,
        scratch_types=[
            pltpu.VMEM((W,), jnp.float32),
            pltpu.VMEM((W,), jnp.int32),
            pltpu.VMEM((W,), jnp.int32),
            pltpu.VMEM((W,), jnp.int32),
            pltpu.VMEM((256,), jnp.int32),
            pltpu.VMEM((NT * 256,), jnp.int32),
            pltpu.VMEM_SHARED((NC * NT * 256,), jnp.int32),
            pltpu.SemaphoreType.DMA,
            pltpu.SemaphoreType.DMA,
        ],
        compiler_params=_params,
    )
    def pass_kernel(*refs):
        if is_p0:
            (xs, *outs, vf, vkey, vidx, vpos, table, gridv, grid,
             sem, sem2) = refs
            src_key = src_idx = None
        else:
            (src_key, src_idx, *outs, vf, vkey, vidx, vpos, table, gridv,
             grid, sem, sem2) = refs
        if write_keys:
            dst_key, dst_idx = outs
        else:
            dst_key, (dst_idx,) = None, outs
        cid = lax.axis_index("c")
        sid = lax.axis_index("s")
        abase = cid * N
        chunk = abase + sid * C
        iota16 = lax.iota(jnp.int32, 16)
        zero16 = jnp.zeros((16,), jnp.int32)
        gbase = cid * (NT * 256)

        def zbody(j, _):
            table[pl.ds(j * 16, 16)] = zero16
            return 0
        lax.fori_loop(0, 16, zbody, 0)

        def hbody(w, _):
            if is_p0:
                pltpu.sync_copy(xs.at[pl.ds(chunk + w * W, W)], vf)
            else:
                pltpu.sync_copy(src_key.at[pl.ds(chunk + w * W, W)], vkey)

            def ibody(i, _):
                if is_p0:
                    kv = _u32key(vf[pl.ds(i * 16, 16)])
                else:
                    kv = vkey[pl.ds(i * 16, 16)]
                d = _digit(kv, shift)
                cnt, last = plsc.scan_count(d)
                cur = plsc.load_gather(table, [d])
                plsc.store_scatter(table, [d], cur + cnt, mask=last)
                return 0
            lax.fori_loop(0, V, ibody, 0)
            return 0
        lax.fori_loop(0, NW, hbody, 0)
        pltpu.sync_copy(table, grid.at[pl.ds(gbase + sid * 256, 256)])
        plsc.subcore_barrier()

        pltpu.sync_copy(grid.at[pl.ds(gbase, NT * 256)], gridv)

        def jbody(j, carry):
            def tbody(t, tp):
                tot, pre = tp
                row = gridv[pl.ds(t * 256 + j * 16, 16)]
                return tot + row, pre + jnp.where(t < sid, row, zero16)
            tot, pre = lax.fori_loop(0, NT, tbody, (zero16, zero16))
            incl = plsc.cumsum(tot)
            table[pl.ds(j * 16, 16)] = incl - tot + carry + pre + abase
            return carry + jnp.sum(tot)
        lax.fori_loop(0, 16, jbody, jnp.int32(0))

        def pbody(w, _):
            base_g = chunk + w * W
            if is_p0:
                pltpu.sync_copy(xs.at[pl.ds(base_g, W)], vf)
            else:
                pltpu.sync_copy(src_key.at[pl.ds(base_g, W)], vkey)
                pltpu.sync_copy(src_idx.at[pl.ds(base_g, W)], vidx)

            def ibody(i, _):
                if is_p0:
                    kv = _u32key(vf[pl.ds(i * 16, 16)])
                    vkey[pl.ds(i * 16, 16)] = kv
                    vidx[pl.ds(i * 16, 16)] = (
                        (base_g - abase) + i * 16 + iota16)
                else:
                    kv = vkey[pl.ds(i * 16, 16)]
                d = _digit(kv, shift)
                cnt, last = plsc.scan_count(d)
                base_o = plsc.load_gather(table, [d])
                vpos[pl.ds(i * 16, 16)] = base_o + cnt - 1
                plsc.store_scatter(table, [d], base_o + cnt, mask=last)
                return 0
            lax.fori_loop(0, V, ibody, 0)
            if write_keys:
                cp1 = pltpu.async_copy(vkey, dst_key.at[vpos], sem)
                cp2 = pltpu.async_copy(vidx, dst_idx.at[vpos], sem2)
                cp1.wait()
                cp2.wait()
            else:
                pltpu.async_copy(vidx, dst_idx.at[vpos], sem).wait()
            return 0
        lax.fori_loop(0, NW, pbody, 0)

    return pass_kernel


_pass0 = _make_pass(0, True, True)
_pass1 = _make_pass(8, False, True)
_pass2 = _make_pass(16, False, True)
_pass3 = _make_pass(24, False, False)


@functools.partial(
    pl.kernel, mesh=mesh, out_type=_f32((2 * N,)),
    scratch_types=[
        pltpu.VMEM((W,), jnp.float32),
        pltpu.VMEM((W,), jnp.int32),
        pltpu.VMEM((W,), jnp.int32),
        pltpu.SemaphoreType.DMA,
    ],
    compiler_params=_params,
)
def _rank_kernel(sigma, rank, vf, vidx, vpos, sem):
    cid = lax.axis_index("c")
    sid = lax.axis_index("s")
    abase = cid * N
    chunk = abase + sid * C
    iota16 = lax.iota(jnp.int32, 16)

    def wbody(w, _):
        base_g = chunk + w * W
        pltpu.sync_copy(sigma.at[pl.ds(base_g, W)], vidx)

        def ibody(i, _):
            sv = vidx[pl.ds(i * 16, 16)]
            vpos[pl.ds(i * 16, 16)] = sv + abase
            vf[pl.ds(i * 16, 16)] = (
                (base_g - abase) + i * 16 + iota16).astype(jnp.float32)
            return 0
        lax.fori_loop(0, V, ibody, 0)
        pltpu.async_copy(vf, rank.at[vpos], sem).wait()
        return 0
    lax.fori_loop(0, NW, wbody, 0)


C2 = N // (NC * NT)
NW2 = C2 // W


@functools.partial(
    pl.kernel, mesh=mesh, out_type=_f32((NC * NT * 16,)),
    scratch_types=[
        pltpu.VMEM((W,), jnp.float32),
        pltpu.VMEM((W,), jnp.float32),
        pltpu.VMEM((16,), jnp.float32),
    ],
    compiler_params=_params,
)
def _dot_kernel(rank, parts, vr0, vr1, vacc):
    cid = lax.axis_index("c")
    sid = lax.axis_index("s")
    wid = cid * NT + sid
    base = wid * C2
    m = jnp.full((16,), MEAN, jnp.float32)

    def wbody(w, acc):
        pltpu.sync_copy(rank.at[pl.ds(base + w * W, W)], vr0)
        pltpu.sync_copy(rank.at[pl.ds(N + base + w * W, W)], vr1)

        def ibody(i, a):
            r0 = vr0[pl.ds(i * 16, 16)] - m
            r1 = vr1[pl.ds(i * 16, 16)] - m
            return a + r0 * r1
        return lax.fori_loop(0, V, ibody, acc)
    acc = lax.fori_loop(0, NW2, wbody, jnp.zeros((16,), jnp.float32))
    vacc[...] = acc
    pltpu.sync_copy(vacc, parts.at[pl.ds(wid * 16, 16)])


_STD32 = np.float32(np.sqrt(np.float64(N) * np.float64(N + 1) / 12.0))
_DENOM = np.float32(np.float32(_STD32 * _STD32) + np.float32(1e-6))


def kernel(predictions, targets):
    xs = jnp.concatenate([predictions, targets])
    kA, iA = _pass0(xs)
    kB, iB = _pass1(kA, iA)
    kA2, iA2 = _pass2(kB, iB)
    sigma, = _pass3(kA2, iA2)
    rank = _rank_kernel(sigma)
    parts = _dot_kernel(rank)
    cov = jnp.sum(parts) / np.float32(N)
    return -(cov / _DENOM)


# speedup vs baseline: 1.7986x; 1.7986x over previous
import functools

import jax
import jax.numpy as jnp
import numpy as np
from jax import lax
from jax.experimental import pallas as pl
from jax.experimental.pallas import tpu as pltpu
from jax.experimental.pallas import tpu_sc as plsc

N = 1048576
NT = 16
NC = 2
C = N // NT
W = 8192
NW = C // W
V = W // 16
B = 2048
MEAN = (N - 1) / 2.0

mesh = plsc.VectorSubcoreMesh(core_axis_name="c", subcore_axis_name="s",
                              num_cores=NC, num_subcores=NT)
_params = pltpu.CompilerParams(needs_layout_passes=False)

_i32 = functools.partial(jax.ShapeDtypeStruct, dtype=jnp.int32)
_f32 = functools.partial(jax.ShapeDtypeStruct, dtype=jnp.float32)


def _u32key(xv):
    b = plsc.bitcast(xv, jnp.int32)
    return jnp.where(b < 0, jnp.bitwise_xor(b, jnp.int32(-1)),
                     jnp.bitwise_or(b, jnp.int32(-(2 ** 31))))


def _digit(kv, shift, bits):
    s = jnp.full((16,), shift, jnp.int32)
    return jnp.bitwise_and(lax.shift_right_logical(kv, s),
                           jnp.full((16,), (1 << bits) - 1, jnp.int32))


def _make_pass(shift, bits, is_p0, is_last):
    out_type = [_f32((2 * N,))] if is_last else [_i32((2 * N,))] * 2

    @functools.partial(
        pl.kernel, mesh=mesh, out_type=out_type,
        scratch_types=[
            pltpu.VMEM((W,), jnp.float32),
            pltpu.VMEM((W,), jnp.int32),
            pltpu.VMEM((W,), jnp.int32),
            pltpu.VMEM((W,), jnp.int32),
            pltpu.VMEM((B,), jnp.int32),
            pltpu.VMEM((NT * B,), jnp.int32),
            pltpu.VMEM_SHARED((NC * NT * B,), jnp.int32),
            pltpu.SemaphoreType.DMA,
            pltpu.SemaphoreType.DMA,
        ],
        compiler_params=_params,
    )
    def pass_kernel(*refs):
        if is_p0:
            (xs, *outs, vf, vkey, vidx, vpos, table, gridv, grid,
             sem, sem2) = refs
            src_key = src_idx = None
        else:
            (src_key, src_idx, *outs, vf, vkey, vidx, vpos, table, gridv,
             grid, sem, sem2) = refs
        if is_last:
            (rank,) = outs
            dst_key = dst_idx = None
        else:
            dst_key, dst_idx = outs
        cid = lax.axis_index("c")
        sid = lax.axis_index("s")
        abase = cid * N
        chunk = abase + sid * C
        iota16 = lax.iota(jnp.int32, 16)
        zero16 = jnp.zeros((16,), jnp.int32)
        gbase = cid * (NT * B)

        def zbody(j, _):
            table[pl.ds(j * 16, 16)] = zero16
            return 0
        lax.fori_loop(0, B // 16, zbody, 0)

        def hbody(w, _):
            if is_p0:
                pltpu.sync_copy(xs.at[pl.ds(chunk + w * W, W)], vf)
            else:
                pltpu.sync_copy(src_key.at[pl.ds(chunk + w * W, W)], vkey)

            def ibody(i, _):
                if is_p0:
                    kv = _u32key(vf[pl.ds(i * 16, 16)])
                else:
                    kv = vkey[pl.ds(i * 16, 16)]
                d = _digit(kv, shift, bits)
                cnt, last = plsc.scan_count(d)
                cur = plsc.load_gather(table, [d])
                plsc.store_scatter(table, [d], cur + cnt, mask=last)
                return 0
            lax.fori_loop(0, V, ibody, 0)
            return 0
        lax.fori_loop(0, NW, hbody, 0)
        pltpu.sync_copy(table, grid.at[pl.ds(gbase + sid * B, B)])
        plsc.subcore_barrier()

        pltpu.sync_copy(grid.at[pl.ds(gbase, NT * B)], gridv)

        def jbody(j, carry):
            def tbody(t, tp):
                tot, pre = tp
                row = gridv[pl.ds(t * B + j * 16, 16)]
                return tot + row, pre + jnp.where(t < sid, row, zero16)
            tot, pre = lax.fori_loop(0, NT, tbody, (zero16, zero16))
            incl = plsc.cumsum(tot)
            base = incl - tot + carry + pre
            table[pl.ds(j * 16, 16)] = base if is_last else base + abase
            return carry + jnp.sum(tot)
        lax.fori_loop(0, B // 16, jbody, jnp.int32(0))

        def pbody(w, _):
            base_g = chunk + w * W
            if is_p0:
                pltpu.sync_copy(xs.at[pl.ds(base_g, W)], vf)
            else:
                pltpu.sync_copy(src_key.at[pl.ds(base_g, W)], vkey)
                pltpu.sync_copy(src_idx.at[pl.ds(base_g, W)], vidx)

            def ibody(i, _):
                if is_p0:
                    kv = _u32key(vf[pl.ds(i * 16, 16)])
                    vkey[pl.ds(i * 16, 16)] = kv
                    vidx[pl.ds(i * 16, 16)] = (
                        (base_g - abase) + i * 16 + iota16)
                else:
                    kv = vkey[pl.ds(i * 16, 16)]
                d = _digit(kv, shift, bits)
                cnt, last = plsc.scan_count(d)
                base_o = plsc.load_gather(table, [d])
                pos = base_o + cnt - 1
                plsc.store_scatter(table, [d], base_o + cnt, mask=last)
                if is_last:
                    vf[pl.ds(i * 16, 16)] = pos.astype(jnp.float32)
                    vpos[pl.ds(i * 16, 16)] = (
                        vidx[pl.ds(i * 16, 16)] + abase)
                else:
                    vpos[pl.ds(i * 16, 16)] = pos
                return 0
            lax.fori_loop(0, V, ibody, 0)
            if is_last:
                pltpu.async_copy(vf, rank.at[vpos], sem).wait()
            else:
                cp1 = pltpu.async_copy(vkey, dst_key.at[vpos], sem)
                cp2 = pltpu.async_copy(vidx, dst_idx.at[vpos], sem2)
                cp1.wait()
                cp2.wait()
            return 0
        lax.fori_loop(0, NW, pbody, 0)

    return pass_kernel


_pass0 = _make_pass(0, 11, True, False)
_pass1 = _make_pass(11, 11, False, False)
_pass2 = _make_pass(22, 10, False, True)


C2 = N // (NC * NT)
NW2 = C2 // W


@functools.partial(
    pl.kernel, mesh=mesh, out_type=_f32((NC * NT * 16,)),
    scratch_types=[
        pltpu.VMEM((W,), jnp.float32),
        pltpu.VMEM((W,), jnp.float32),
        pltpu.VMEM((16,), jnp.float32),
    ],
    compiler_params=_params,
)
def _dot_kernel(rank, parts, vr0, vr1, vacc):
    cid = lax.axis_index("c")
    sid = lax.axis_index("s")
    wid = cid * NT + sid
    base = wid * C2
    m = jnp.full((16,), MEAN, jnp.float32)

    def wbody(w, acc):
        pltpu.sync_copy(rank.at[pl.ds(base + w * W, W)], vr0)
        pltpu.sync_copy(rank.at[pl.ds(N + base + w * W, W)], vr1)

        def ibody(i, a):
            r0 = vr0[pl.ds(i * 16, 16)] - m
            r1 = vr1[pl.ds(i * 16, 16)] - m
            return a + r0 * r1
        return lax.fori_loop(0, V, ibody, acc)
    acc = lax.fori_loop(0, NW2, wbody, jnp.zeros((16,), jnp.float32))
    vacc[...] = acc
    pltpu.sync_copy(vacc, parts.at[pl.ds(wid * 16, 16)])


_STD32 = np.float32(np.sqrt(np.float64(N) * np.float64(N + 1) / 12.0))
_DENOM = np.float32(np.float32(_STD32 * _STD32) + np.float32(1e-6))


def kernel(predictions, targets):
    xs = jnp.concatenate([predictions, targets])
    kA, iA = _pass0(xs)
    kB, iB = _pass1(kA, iA)
    rank, = _pass2(kB, iB)
    parts = _dot_kernel(rank)
    cov = jnp.sum(parts) / np.float32(N)
    return -(cov / _DENOM)
